# baseline (speedup 1.0000x reference)
import jax
import jax.numpy as jnp
from jax.experimental import pallas as pl

_N = 32768
_B = 16
_EPS = 1e-5


def _body(x_ref, starts_ref, ends_ref, lenf_ref,
          W1_ref, b1_ref, g1_ref, be1_ref,
          W2c_ref, b2_ref, g2_ref, be2_ref,
          Wfc1_ref, bfc1_ref, gfc1_ref, befc1_ref,
          Wfc2_ref, bfc2_ref, gfc2_ref, befc2_ref,
          out_ref):
    x = x_ref[...]
    f32 = jnp.float32
    bf16 = jnp.bfloat16
    xb = x.astype(bf16)

    s1t = jax.lax.dot_general(W1_ref[...].astype(bf16), xb,
                              (((1,), (1,)), ((), ())),
                              preferred_element_type=f32)
    s1t = s1t + b1_ref[...]
    m1 = jnp.mean(s1t, axis=1, keepdims=True)
    c1 = s1t - m1
    v1 = jnp.mean(jnp.square(c1), axis=1, keepdims=True)
    o1 = c1 * jax.lax.rsqrt(v1 + _EPS) * g1_ref[...] + be1_ref[...]
    o1 = jnp.maximum(o1, 0.0)

    w2c = W2c_ref[...].astype(bf16).astype(f32)
    s2 = jnp.sum(o1.astype(bf16).astype(f32) * w2c, axis=0,
                 keepdims=True) + b2_ref[...]
    m2 = jnp.mean(s2, axis=1, keepdims=True)
    c2 = s2 - m2
    v2 = jnp.mean(jnp.square(c2), axis=1, keepdims=True)
    att = c2 * jax.lax.rsqrt(v2 + _EPS) * g2_ref[...] + be2_ref[...]

    idx = jax.lax.broadcasted_iota(jnp.int32, (1, _N), 1)
    inseg = (idx >= starts_ref[...]) & (idx < ends_ref[...])

    neg = jnp.full((_B, _N), -jnp.inf, f32)
    seg_max = jnp.max(jnp.where(inseg, att, neg), axis=1, keepdims=True)
    delta = jnp.minimum(att - seg_max, 0.0)
    e = jnp.where(inseg, jnp.exp(delta), 0.0)
    seg_sum = jnp.sum(e, axis=1, keepdims=True)
    w = e / seg_sum

    acc = jax.lax.dot_general(w.astype(bf16), xb, (((1,), (0,)), ((), ())),
                              preferred_element_type=f32)
    r = acc / lenf_ref[...]

    z1 = jax.lax.dot_general(r.astype(bf16), Wfc1_ref[...].astype(bf16),
                             (((1,), (1,)), ((), ())),
                             preferred_element_type=f32) + bfc1_ref[...]
    mz1 = jnp.mean(z1, axis=0, keepdims=True)
    cz1 = z1 - mz1
    vz1 = jnp.mean(jnp.square(cz1), axis=0, keepdims=True)
    h1 = cz1 * jax.lax.rsqrt(vz1 + _EPS) * gfc1_ref[...] + befc1_ref[...]
    h1 = jnp.maximum(h1, 0.0)

    z2 = jax.lax.dot_general(h1.astype(bf16), Wfc2_ref[...].astype(bf16),
                             (((1,), (1,)), ((), ())),
                             preferred_element_type=f32) + bfc2_ref[...]
    mz2 = jnp.mean(z2, axis=0, keepdims=True)
    cz2 = z2 - mz2
    vz2 = jnp.mean(jnp.square(cz2), axis=0, keepdims=True)
    h2 = cz2 * jax.lax.rsqrt(vz2 + _EPS) * gfc2_ref[...] + befc2_ref[...]

    nrm = jnp.sqrt(jnp.sum(jnp.square(h2), axis=1, keepdims=True))
    out_ref[...] = h2 / jnp.maximum(nrm, 1e-12)


def kernel(x, length, W1, b1, g1, be1, W2, b2, g2, be2,
           Wfc1, bfc1, gfc1, befc1, Wfc2, bfc2, gfc2, befc2):
    length = length.astype(jnp.int32)
    ends = jnp.cumsum(length)
    starts = ends - length
    args = (
        x,
        starts.reshape(_B, 1), ends.reshape(_B, 1),
        length.astype(jnp.float32).reshape(_B, 1),
        W1, b1.reshape(16, 1), g1.reshape(16, 1), be1.reshape(16, 1),
        W2.reshape(16, 1), b2.reshape(1, 1), g2.reshape(1, 1), be2.reshape(1, 1),
        Wfc1, bfc1.reshape(1, 64), gfc1.reshape(1, 64), befc1.reshape(1, 64),
        Wfc2, bfc2.reshape(1, 256), gfc2.reshape(1, 256), befc2.reshape(1, 256),
    )
    return pl.pallas_call(
        _body,
        out_shape=jax.ShapeDtypeStruct((_B, 256), jnp.float32),
    )(*args)
